# baseline (speedup 1.0000x reference)
import functools

import jax
import jax.numpy as jnp
from jax.experimental import pallas as pl
from jax.experimental.pallas import tpu as pltpu


def _round_up(x, m):
    return (x + m - 1) // m * m


def _gather_block_kernel(ids_smem, idv_ref, table_ref, out_ref, sem, *,
                         vocab_size, block_rows):
    base = pl.program_id(0) * block_rows
    for w in range(block_rows):
        raw_id = ids_smem[base + w]
        safe_id = jnp.minimum(jnp.maximum(raw_id, 0), vocab_size - 1)
        pltpu.make_async_copy(
            table_ref.at[pl.ds(safe_id, 1)],
            out_ref.at[pl.ds(w, 1)],
            sem,
        ).start(priority=w & 1)
    pltpu.make_async_copy(
        table_ref.at[pl.ds(0, block_rows)],
        out_ref.at[pl.ds(0, block_rows)],
        sem,
    ).wait()
    ids_v = idv_ref[0]
    valid = jnp.logical_and(ids_v >= 0, ids_v < vocab_size)
    out_ref[...] = jnp.where(valid, out_ref[...], jnp.float32(0.0))


@functools.partial(jax.jit, static_argnames=("block_rows",))
def _w2v_gather(word_ids, embedding_table, block_rows=512):
    n = word_ids.shape[0]
    v, d = embedding_table.shape
    ids = word_ids.astype(jnp.int32)
    n_pad = _round_up(n, block_rows)
    if n_pad != n:
        ids = jnp.pad(ids, (0, n_pad - n), constant_values=-1)
    n_blocks = n_pad // block_rows
    ids_vec = ids.reshape(n_blocks, block_rows, 1)

    kernel_fn = functools.partial(
        _gather_block_kernel, vocab_size=v, block_rows=block_rows)
    out = pl.pallas_call(
        kernel_fn,
        out_shape=jax.ShapeDtypeStruct((n_pad, d), embedding_table.dtype),
        grid_spec=pltpu.PrefetchScalarGridSpec(
            num_scalar_prefetch=1,
            grid=(n_blocks,),
            in_specs=[
                pl.BlockSpec((1, block_rows, 1), lambda i, ids_r: (i, 0, 0)),
                pl.BlockSpec(memory_space=pl.ANY),
            ],
            out_specs=pl.BlockSpec((block_rows, d), lambda i, ids_r: (i, 0)),
            scratch_shapes=[pltpu.SemaphoreType.DMA],
        ),
        compiler_params=pltpu.CompilerParams(
            dimension_semantics=("parallel",),
        ),
    )(ids, ids_vec, embedding_table)
    if n_pad != n:
        out = out[:n]
    return out


def kernel(word_ids, embedding_table):
    return _w2v_gather(word_ids, embedding_table, block_rows=512)


# speedup vs baseline: 1.1192x; 1.1192x over previous
import functools

import jax
import jax.numpy as jnp
from jax.experimental import pallas as pl
from jax.experimental.pallas import tpu as pltpu


def _round_up(x, m):
    return (x + m - 1) // m * m


def _gather_block_kernel(ids_smem, idv_ref, table_ref, out_ref, sem, *,
                         vocab_size, block_rows):
    base = pl.program_id(0) * block_rows
    for w in range(block_rows):
        raw_id = ids_smem[base + w]
        safe_id = jnp.minimum(jnp.maximum(raw_id, 0), vocab_size - 1)
        pltpu.make_async_copy(
            table_ref.at[pl.ds(safe_id, 1)],
            out_ref.at[pl.ds(w, 1)],
            sem,
        ).start()
    pltpu.make_async_copy(
        table_ref.at[pl.ds(0, block_rows)],
        out_ref.at[pl.ds(0, block_rows)],
        sem,
    ).wait()
    ids_v = idv_ref[0]
    valid = jnp.logical_and(ids_v >= 0, ids_v < vocab_size)
    out_ref[...] = jnp.where(valid, out_ref[...], jnp.float32(0.0))


@functools.partial(jax.jit, static_argnames=("block_rows",))
def _w2v_gather(word_ids, embedding_table, block_rows=512):
    n = word_ids.shape[0]
    v, d = embedding_table.shape
    ids = word_ids.astype(jnp.int32)
    n_pad = _round_up(n, block_rows)
    if n_pad != n:
        ids = jnp.pad(ids, (0, n_pad - n), constant_values=-1)
    n_blocks = n_pad // block_rows
    ids_vec = ids.reshape(n_blocks, block_rows, 1)

    kernel_fn = functools.partial(
        _gather_block_kernel, vocab_size=v, block_rows=block_rows)
    out = pl.pallas_call(
        kernel_fn,
        out_shape=jax.ShapeDtypeStruct((n_pad, d), embedding_table.dtype),
        grid_spec=pltpu.PrefetchScalarGridSpec(
            num_scalar_prefetch=1,
            grid=(n_blocks,),
            in_specs=[
                pl.BlockSpec((1, block_rows, 1), lambda i, ids_r: (i, 0, 0)),
                pl.BlockSpec(memory_space=pl.ANY),
            ],
            out_specs=pl.BlockSpec((block_rows, d), lambda i, ids_r: (i, 0)),
            scratch_shapes=[pltpu.SemaphoreType.DMA],
        ),
        compiler_params=pltpu.CompilerParams(
            dimension_semantics=("parallel",),
        ),
    )(ids, ids_vec, embedding_table)
    if n_pad != n:
        out = out[:n]
    return out


def kernel(word_ids, embedding_table):
    return _w2v_gather(word_ids, embedding_table, block_rows=1024)


# speedup vs baseline: 1.1797x; 1.0541x over previous
import functools

import jax
import jax.numpy as jnp
from jax.experimental import pallas as pl
from jax.experimental.pallas import tpu as pltpu


def _round_up(x, m):
    return (x + m - 1) // m * m


def _gather_block_kernel(ids_smem, idv_ref, table_ref, out_ref, sem, *,
                         vocab_size, block_rows):
    base = pl.program_id(0) * block_rows
    for w in range(block_rows):
        raw_id = ids_smem[base + w]
        safe_id = jnp.minimum(jnp.maximum(raw_id, 0), vocab_size - 1)
        pltpu.make_async_copy(
            table_ref.at[pl.ds(safe_id, 1)],
            out_ref.at[pl.ds(w, 1)],
            sem,
        ).start()
    pltpu.make_async_copy(
        table_ref.at[pl.ds(0, block_rows)],
        out_ref.at[pl.ds(0, block_rows)],
        sem,
    ).wait()
    ids_v = idv_ref[0]
    valid = jnp.logical_and(ids_v >= 0, ids_v < vocab_size)
    out_ref[...] = jnp.where(valid, out_ref[...], jnp.float32(0.0))


@functools.partial(jax.jit, static_argnames=("block_rows",))
def _w2v_gather(word_ids, embedding_table, block_rows=512):
    n = word_ids.shape[0]
    v, d = embedding_table.shape
    ids = word_ids.astype(jnp.int32)
    n_pad = _round_up(n, block_rows)
    if n_pad != n:
        ids = jnp.pad(ids, (0, n_pad - n), constant_values=-1)
    n_blocks = n_pad // block_rows
    ids_vec = ids.reshape(n_blocks, block_rows, 1)

    kernel_fn = functools.partial(
        _gather_block_kernel, vocab_size=v, block_rows=block_rows)
    out = pl.pallas_call(
        kernel_fn,
        out_shape=jax.ShapeDtypeStruct((n_pad, d), embedding_table.dtype),
        grid_spec=pltpu.PrefetchScalarGridSpec(
            num_scalar_prefetch=1,
            grid=(n_blocks,),
            in_specs=[
                pl.BlockSpec((1, block_rows, 1), lambda i, ids_r: (i, 0, 0)),
                pl.BlockSpec(memory_space=pl.ANY),
            ],
            out_specs=pl.BlockSpec((block_rows, d), lambda i, ids_r: (i, 0)),
            scratch_shapes=[pltpu.SemaphoreType.DMA],
        ),
        compiler_params=pltpu.CompilerParams(
            dimension_semantics=("parallel",),
        ),
    )(ids, ids_vec, embedding_table)
    if n_pad != n:
        out = out[:n]
    return out


def kernel(word_ids, embedding_table):
    return _w2v_gather(word_ids, embedding_table, block_rows=2048)


# speedup vs baseline: 1.1960x; 1.0138x over previous
import functools

import jax
import jax.numpy as jnp
from jax.experimental import pallas as pl
from jax.experimental.pallas import tpu as pltpu


def _round_up(x, m):
    return (x + m - 1) // m * m


def _gather_block_kernel(ids_smem, idv_ref, table_ref, out_ref, sem, *,
                         vocab_size, block_rows):
    base = pl.program_id(0) * block_rows
    for w in range(block_rows):
        raw_id = ids_smem[base + w]
        safe_id = jnp.minimum(jnp.maximum(raw_id, 0), vocab_size - 1)
        pltpu.make_async_copy(
            table_ref.at[pl.ds(safe_id, 1)],
            out_ref.at[pl.ds(w, 1)],
            sem,
        ).start()
    pltpu.make_async_copy(
        table_ref.at[pl.ds(0, block_rows)],
        out_ref.at[pl.ds(0, block_rows)],
        sem,
    ).wait()
    ids_v = idv_ref[0]
    valid = jnp.logical_and(ids_v >= 0, ids_v < vocab_size)
    out_ref[...] = jnp.where(valid, out_ref[...], jnp.float32(0.0))


@functools.partial(jax.jit, static_argnames=("block_rows",))
def _w2v_gather(word_ids, embedding_table, block_rows=512):
    n = word_ids.shape[0]
    v, d = embedding_table.shape
    ids = word_ids.astype(jnp.int32)
    n_pad = _round_up(n, block_rows)
    if n_pad != n:
        ids = jnp.pad(ids, (0, n_pad - n), constant_values=-1)
    n_blocks = n_pad // block_rows
    ids_vec = ids.reshape(n_blocks, block_rows, 1)

    kernel_fn = functools.partial(
        _gather_block_kernel, vocab_size=v, block_rows=block_rows)
    out = pl.pallas_call(
        kernel_fn,
        out_shape=jax.ShapeDtypeStruct((n_pad, d), embedding_table.dtype),
        grid_spec=pltpu.PrefetchScalarGridSpec(
            num_scalar_prefetch=1,
            grid=(n_blocks,),
            in_specs=[
                pl.BlockSpec((1, block_rows, 1), lambda i, ids_r: (i, 0, 0)),
                pl.BlockSpec(memory_space=pl.ANY),
            ],
            out_specs=pl.BlockSpec((block_rows, d), lambda i, ids_r: (i, 0)),
            scratch_shapes=[pltpu.SemaphoreType.DMA],
        ),
        compiler_params=pltpu.CompilerParams(
            dimension_semantics=("parallel",),
        ),
    )(ids, ids_vec, embedding_table)
    if n_pad != n:
        out = out[:n]
    return out


def kernel(word_ids, embedding_table):
    return _w2v_gather(word_ids, embedding_table, block_rows=4096)


# speedup vs baseline: 1.2567x; 1.0508x over previous
import functools

import jax
import jax.numpy as jnp
from jax.experimental import pallas as pl
from jax.experimental.pallas import tpu as pltpu


def _round_up(x, m):
    return (x + m - 1) // m * m


def _gather_block_kernel(ids_smem, idv_ref, table_ref, out_ref, sem, *,
                         vocab_size, block_rows):
    base = pl.program_id(0) * block_rows
    for w in range(block_rows):
        safe_id = ids_smem[base + w]
        pltpu.make_async_copy(
            table_ref.at[pl.ds(safe_id, 1)],
            out_ref.at[pl.ds(w, 1)],
            sem,
        ).start()
    pltpu.make_async_copy(
        table_ref.at[pl.ds(0, block_rows)],
        out_ref.at[pl.ds(0, block_rows)],
        sem,
    ).wait()
    ids_v = idv_ref[0]
    valid = jnp.logical_and(ids_v >= 0, ids_v < vocab_size)
    out_ref[...] = jnp.where(valid, out_ref[...], jnp.float32(0.0))


@functools.partial(jax.jit, static_argnames=("block_rows",))
def _w2v_gather(word_ids, embedding_table, block_rows=512):
    n = word_ids.shape[0]
    v, d = embedding_table.shape
    ids = word_ids.astype(jnp.int32)
    n_pad = _round_up(n, block_rows)
    if n_pad != n:
        ids = jnp.pad(ids, (0, n_pad - n), constant_values=-1)
    n_blocks = n_pad // block_rows
    ids_vec = ids.reshape(n_blocks, block_rows, 1)
    ids_safe = jnp.clip(ids, 0, v - 1)

    kernel_fn = functools.partial(
        _gather_block_kernel, vocab_size=v, block_rows=block_rows)
    out = pl.pallas_call(
        kernel_fn,
        out_shape=jax.ShapeDtypeStruct((n_pad, d), embedding_table.dtype),
        grid_spec=pltpu.PrefetchScalarGridSpec(
            num_scalar_prefetch=1,
            grid=(n_blocks,),
            in_specs=[
                pl.BlockSpec((1, block_rows, 1), lambda i, ids_r: (i, 0, 0)),
                pl.BlockSpec(memory_space=pl.ANY),
            ],
            out_specs=pl.BlockSpec((block_rows, d), lambda i, ids_r: (i, 0)),
            scratch_shapes=[pltpu.SemaphoreType.DMA],
        ),
        compiler_params=pltpu.CompilerParams(
            dimension_semantics=("parallel",),
        ),
    )(ids_safe, ids_vec, embedding_table)
    if n_pad != n:
        out = out[:n]
    return out


def kernel(word_ids, embedding_table):
    return _w2v_gather(word_ids, embedding_table, block_rows=4096)
